# speedup vs baseline: 1.0091x; 1.0091x over previous
import functools

import jax
import jax.numpy as jnp
from jax.experimental import pallas as pl
from jax.experimental.pallas import tpu as pltpu

N = 50000
ROW_BLK = 1000


def _dense_body(ego_ref, side_ref, W1_ref, b1_ref, W2_ref, b2_ref,
                g1_ref, be1_ref, g2_ref, be2_ref,
                ego_out_ref, nrm_out_ref):
    ego = ego_ref[...]
    side = side_ref[...]
    W1 = W1_ref[...]
    W2 = W2_ref[...]

    h1 = jnp.dot((ego + side), W1, preferred_element_type=jnp.float32) + b1_ref[...]
    h1 = jnp.where(h1 > 0, h1, 0.01 * h1)
    mu1 = jnp.mean(h1, axis=-1, keepdims=True)
    var1 = jnp.mean((h1 - mu1) ** 2, axis=-1, keepdims=True)
    s1 = (h1 - mu1) * jax.lax.rsqrt(var1 + 1e-5) * g1_ref[...] + be1_ref[...]

    h2 = jnp.dot((ego * side), W2, preferred_element_type=jnp.float32) + b2_ref[...]
    h2 = jnp.where(h2 > 0, h2, 0.01 * h2)
    mu2 = jnp.mean(h2, axis=-1, keepdims=True)
    var2 = jnp.mean((h2 - mu2) ** 2, axis=-1, keepdims=True)
    s2 = (h2 - mu2) * jax.lax.rsqrt(var2 + 1e-5) * g2_ref[...] + be2_ref[...]

    out = s1 + s2
    ego_out_ref[...] = out
    nrm = jnp.sqrt(jnp.sum(out * out, axis=-1, keepdims=True))
    nrm_out_ref[...] = out / jnp.maximum(nrm, 1e-12)


def _dense_stage(ego, side, W1, b1, W2, b2, g1, be1, g2, be2):
    din = ego.shape[1]
    dout = W1.shape[1]
    grid = (N // ROW_BLK,)
    row_spec = pl.BlockSpec((ROW_BLK, din), lambda i: (i, 0))
    out_spec = pl.BlockSpec((ROW_BLK, dout), lambda i: (i, 0))
    full = lambda a: pl.BlockSpec(a.shape, lambda i: tuple(0 for _ in a.shape))
    vec = lambda a: pl.BlockSpec((1, dout), lambda i: (0, 0))
    ego_next, nrm = pl.pallas_call(
        _dense_body,
        grid=grid,
        in_specs=[row_spec, row_spec,
                  full(W1), vec(b1), full(W2), vec(b2),
                  vec(g1), vec(be1), vec(g2), vec(be2)],
        out_specs=[out_spec, out_spec],
        out_shape=[jax.ShapeDtypeStruct((N, dout), jnp.float32),
                   jax.ShapeDtypeStruct((N, dout), jnp.float32)],
    )(ego, side, W1, b1.reshape(1, -1), W2, b2.reshape(1, -1),
      g1.reshape(1, -1), be1.reshape(1, -1), g2.reshape(1, -1), be2.reshape(1, -1))
    return ego_next, nrm


def kernel(x, edge_index, edge_weight,
           W1_0, b1_0, W2_0, b2_0, g1_0, be1_0, g2_0, be2_0,
           W1_1, b1_1, W2_1, b2_1, g1_1, be1_1, g2_1, be2_1):
    dst = edge_index[0]
    src = edge_index[1]
    params = [
        (W1_0, b1_0, W2_0, b2_0, g1_0, be1_0, g2_0, be2_0),
        (W1_1, b1_1, W2_1, b2_1, g1_1, be1_1, g2_1, be2_1),
    ]
    ego = x
    outs = [x]
    for p in params:
        msgs = edge_weight[:, None] * jnp.take(ego, src, axis=0)
        side = jax.ops.segment_sum(msgs, dst, num_segments=N)
        ego, nrm = _dense_stage(ego, side, *p)
        outs.append(nrm)
    return jnp.concatenate(outs, axis=1)


# speedup vs baseline: 2.6952x; 2.6709x over previous
import functools

import jax
import jax.numpy as jnp
from jax import lax
from jax.experimental import pallas as pl
from jax.experimental.pallas import tpu as pltpu
from jax.experimental.pallas import tpu_sc as plsc

N = 50000
E = 800000
ROW_BLK = 1000

NC = 2
NS = 16
ROWS_MAIN = 3128
ROWS_LAST = N - 15 * ROWS_MAIN


def _scale_rows(rows_ref, w_ref, nedge):
    def scale_window(base, jlo):
        wv = w_ref[pl.ds(base, 16)]
        for j in range(jlo, 16):
            e = base + j
            w = wv[j]
            r0 = rows_ref[e, pl.ds(0, 16)]
            rows_ref[e, pl.ds(0, 16)] = r0 * w
            r1 = rows_ref[e, pl.ds(16, 16)]
            rows_ref[e, pl.ds(16, 16)] = r1 * w

    for g in range(nedge // 16):
        scale_window(g * 16, 0)
    rem = nedge % 16
    if rem:
        scale_window(nedge - 16, 16 - rem)


def _make_sc_agg(chunk, nchunk, l0_split):
    mesh = plsc.VectorSubcoreMesh(core_axis_name="c", subcore_axis_name="s")

    @functools.partial(
        pl.kernel,
        out_type=jax.ShapeDtypeStruct((NC, N, 32), jnp.float32),
        mesh=mesh,
        scratch_types=[
            pltpu.VMEM((chunk,), jnp.int32),
            pltpu.VMEM((chunk,), jnp.int32),
            pltpu.VMEM((chunk,), jnp.float32),
            pltpu.VMEM((chunk, 32), jnp.float32),
            pltpu.VMEM_SHARED((N, 32), jnp.float32),
            pltpu.SemaphoreType.DMA,
        ],
        compiler_params=pltpu.CompilerParams(use_tc_tiling_on_sc=False),
    )
    def agg(tabA, tabB, src, dst, w, zeros, out, src_v, dst_v, w_v, rows_v, acc, sem):
        c = lax.axis_index("c")
        s = lax.axis_index("s")

        zbase = s * ROWS_MAIN

        @pl.when(s < 15)
        def _():
            pltpu.sync_copy(zeros.at[pl.ds(zbase, ROWS_MAIN)],
                            acc.at[pl.ds(zbase, ROWS_MAIN)])

        @pl.when(s == 15)
        def _():
            pltpu.sync_copy(zeros.at[pl.ds(15 * ROWS_MAIN, ROWS_LAST)],
                            acc.at[pl.ds(15 * ROWS_MAIN, ROWS_LAST)])

        plsc.subcore_barrier()

        if l0_split:
            tile_base = s * (E // NS)
        else:
            tile_base = c * (E // NC) + s * (E // (NC * NS))

        def body(g, carry):
            base = tile_base + g * chunk
            pltpu.sync_copy(src.at[pl.ds(base, chunk)], src_v)
            pltpu.sync_copy(dst.at[pl.ds(base, chunk)], dst_v)
            pltpu.sync_copy(w.at[pl.ds(base, chunk)], w_v)

            @pl.when(c == 0)
            def _():
                pltpu.async_copy(tabA.at[src_v], rows_v, sem).wait()

            @pl.when(c == 1)
            def _():
                pltpu.async_copy(tabB.at[src_v], rows_v, sem).wait()

            _scale_rows(rows_v, w_v, chunk)
            pltpu.sync_copy(rows_v, acc.at[dst_v], add=True)
            return carry

        lax.fori_loop(0, nchunk, body, 0)
        plsc.subcore_barrier()

        @pl.when(s < 15)
        def _():
            pltpu.sync_copy(acc.at[pl.ds(zbase, ROWS_MAIN)],
                            out.at[c, pl.ds(zbase, ROWS_MAIN)])

        @pl.when(s == 15)
        def _():
            pltpu.sync_copy(acc.at[pl.ds(15 * ROWS_MAIN, ROWS_LAST)],
                            out.at[c, pl.ds(15 * ROWS_MAIN, ROWS_LAST)])

    return agg


_agg_l0 = _make_sc_agg(chunk=80, nchunk=E // NS // 80, l0_split=True)
_agg_l1 = _make_sc_agg(chunk=40, nchunk=E // (NC * NS) // 40, l0_split=False)


def _dense_body(ego_ref, side_ref, W1_ref, b1_ref, W2_ref, b2_ref,
                g1_ref, be1_ref, g2_ref, be2_ref,
                ego_out_ref, nrm_out_ref, *, side_mode):
    ego = ego_ref[...]
    if side_mode == "concat":
        side = jnp.concatenate([side_ref[0], side_ref[1]], axis=-1)
    else:
        side = side_ref[0] + side_ref[1]

    h1 = jnp.dot((ego + side), W1_ref[...], preferred_element_type=jnp.float32) + b1_ref[...]
    h1 = jnp.where(h1 > 0, h1, 0.01 * h1)
    mu1 = jnp.mean(h1, axis=-1, keepdims=True)
    var1 = jnp.mean((h1 - mu1) ** 2, axis=-1, keepdims=True)
    s1 = (h1 - mu1) * lax.rsqrt(var1 + 1e-5) * g1_ref[...] + be1_ref[...]

    h2 = jnp.dot((ego * side), W2_ref[...], preferred_element_type=jnp.float32) + b2_ref[...]
    h2 = jnp.where(h2 > 0, h2, 0.01 * h2)
    mu2 = jnp.mean(h2, axis=-1, keepdims=True)
    var2 = jnp.mean((h2 - mu2) ** 2, axis=-1, keepdims=True)
    s2 = (h2 - mu2) * lax.rsqrt(var2 + 1e-5) * g2_ref[...] + be2_ref[...]

    out = s1 + s2
    ego_out_ref[...] = out
    nrm = jnp.sqrt(jnp.sum(out * out, axis=-1, keepdims=True))
    nrm_out_ref[...] = out / jnp.maximum(nrm, 1e-12)


def _dense_stage(ego, side2, W1, b1, W2, b2, g1, be1, g2, be2, side_mode):
    din = ego.shape[1]
    dout = W1.shape[1]
    grid = (N // ROW_BLK,)
    row_spec = pl.BlockSpec((ROW_BLK, din), lambda i: (i, 0))
    side_spec = pl.BlockSpec((2, ROW_BLK, 32), lambda i: (0, i, 0))
    out_spec = pl.BlockSpec((ROW_BLK, dout), lambda i: (i, 0))
    full = lambda a: pl.BlockSpec(a.shape, lambda i: tuple(0 for _ in a.shape))
    vec = lambda a: pl.BlockSpec((1, dout), lambda i: (0, 0))
    ego_next, nrm = pl.pallas_call(
        functools.partial(_dense_body, side_mode=side_mode),
        grid=grid,
        in_specs=[row_spec, side_spec,
                  full(W1), vec(b1), full(W2), vec(b2),
                  vec(g1), vec(be1), vec(g2), vec(be2)],
        out_specs=[out_spec, out_spec],
        out_shape=[jax.ShapeDtypeStruct((N, dout), jnp.float32),
                   jax.ShapeDtypeStruct((N, dout), jnp.float32)],
    )(ego, side2, W1, b1.reshape(1, -1), W2, b2.reshape(1, -1),
      g1.reshape(1, -1), be1.reshape(1, -1), g2.reshape(1, -1), be2.reshape(1, -1))
    return ego_next, nrm


def kernel(x, edge_index, edge_weight,
           W1_0, b1_0, W2_0, b2_0, g1_0, be1_0, g2_0, be2_0,
           W1_1, b1_1, W2_1, b2_1, g1_1, be1_1, g2_1, be2_1):
    dst = edge_index[0]
    src = edge_index[1]
    zeros = jnp.zeros((N, 32), jnp.float32)

    x2 = jnp.transpose(x.reshape(N, 2, 32), (1, 0, 2))
    side0 = _agg_l0(x2[0], x2[1], src, dst, edge_weight, zeros)
    ego1, out1 = _dense_stage(x, side0, W1_0, b1_0, W2_0, b2_0,
                              g1_0, be1_0, g2_0, be2_0, side_mode="concat")

    side1 = _agg_l1(ego1, ego1, src, dst, edge_weight, zeros)
    ego2, out2 = _dense_stage(ego1, side1, W1_1, b1_1, W2_1, b2_1,
                              g1_1, be1_1, g2_1, be2_1, side_mode="sum")

    return jnp.concatenate([x, out1, out2], axis=1)


# speedup vs baseline: 4.7868x; 1.7760x over previous
import functools

import jax
import jax.numpy as jnp
from jax import lax
from jax.experimental import pallas as pl
from jax.experimental.pallas import tpu as pltpu
from jax.experimental.pallas import tpu_sc as plsc

N = 50000
E = 800000
ROW_BLK = 1000

NC = 2
NS = 16
CH = 80
CPS = 25
NSUP = 25
CHUNKS_PER_TILE = E // (NS * CH)

ROWS_MAIN = 3128
ROWS_LAST = N - 15 * ROWS_MAIN


def _scale_rows(rows_ref, w_ref, cc, dh):
    for g in range(CH // 16):
        wv = w_ref[cc, pl.ds(g * 16, 16)]
        for j in range(16):
            e = g * 16 + j
            w = wv[j]
            for h in range(dh // 16):
                r = rows_ref[e, pl.ds(h * 16, 16)]
                rows_ref[e, pl.ds(h * 16, 16)] = r * w


def _make_sc_agg(dh):
    mesh = plsc.VectorSubcoreMesh(core_axis_name="c", subcore_axis_name="s")

    @functools.partial(
        pl.kernel,
        out_type=jax.ShapeDtypeStruct((NC, N, dh), jnp.float32),
        mesh=mesh,
        scratch_types=[
            pltpu.VMEM((CPS, CH), jnp.int32),
            pltpu.VMEM((CPS, CH), jnp.int32),
            pltpu.VMEM((CPS, CH), jnp.float32),
            pltpu.VMEM((CH, dh), jnp.float32),
            pltpu.VMEM_SHARED((N, dh), jnp.float32),
            pltpu.SemaphoreType.DMA,
        ],
        compiler_params=pltpu.CompilerParams(use_tc_tiling_on_sc=False),
    )
    def agg(tabA, tabB, src2, dst2, w2, zeros, out,
            src_v, dst_v, w_v, rows_v, acc, gsem):
        c = lax.axis_index("c")
        s = lax.axis_index("s")

        zbase = s * ROWS_MAIN

        @pl.when(s < 15)
        def _():
            pltpu.sync_copy(zeros.at[pl.ds(zbase, ROWS_MAIN)],
                            acc.at[pl.ds(zbase, ROWS_MAIN)])

        @pl.when(s == 15)
        def _():
            pltpu.sync_copy(zeros.at[pl.ds(15 * ROWS_MAIN, ROWS_LAST)],
                            acc.at[pl.ds(15 * ROWS_MAIN, ROWS_LAST)])

        plsc.subcore_barrier()

        tile_row = s * CHUNKS_PER_TILE

        def sup_body(si, carry):
            base = tile_row + si * CPS
            pltpu.sync_copy(src2.at[pl.ds(base, CPS)], src_v)
            pltpu.sync_copy(dst2.at[pl.ds(base, CPS)], dst_v)
            pltpu.sync_copy(w2.at[pl.ds(base, CPS)], w_v)

            def chunk_body(cc, carry2):
                @pl.when(c == 0)
                def _():
                    pltpu.async_copy(tabA.at[src_v.at[cc]], rows_v, gsem).wait()

                @pl.when(c == 1)
                def _():
                    pltpu.async_copy(tabB.at[src_v.at[cc]], rows_v, gsem).wait()

                _scale_rows(rows_v, w_v, cc, dh)
                pltpu.sync_copy(rows_v, acc.at[dst_v.at[cc]], add=True)
                return carry2

            lax.fori_loop(0, CPS, chunk_body, 0)
            return carry

        lax.fori_loop(0, NSUP, sup_body, 0)

        plsc.subcore_barrier()

        @pl.when(s < 15)
        def _():
            pltpu.sync_copy(acc.at[pl.ds(zbase, ROWS_MAIN)],
                            out.at[c, pl.ds(zbase, ROWS_MAIN)])

        @pl.when(s == 15)
        def _():
            pltpu.sync_copy(acc.at[pl.ds(15 * ROWS_MAIN, ROWS_LAST)],
                            out.at[c, pl.ds(15 * ROWS_MAIN, ROWS_LAST)])

    return agg


_agg_l0 = _make_sc_agg(32)
_agg_l1 = _make_sc_agg(16)


def _dense_body(ego_ref, side_ref, W1_ref, b1_ref, W2_ref, b2_ref,
                g1_ref, be1_ref, g2_ref, be2_ref,
                ego_out_ref, nrm_out_ref, *, split_out):
    ego = ego_ref[...]
    side = jnp.concatenate([side_ref[0], side_ref[1]], axis=-1)

    h1 = jnp.dot((ego + side), W1_ref[...], preferred_element_type=jnp.float32) + b1_ref[...]
    h1 = jnp.where(h1 > 0, h1, 0.01 * h1)
    mu1 = jnp.mean(h1, axis=-1, keepdims=True)
    var1 = jnp.mean((h1 - mu1) ** 2, axis=-1, keepdims=True)
    s1 = (h1 - mu1) * lax.rsqrt(var1 + 1e-5) * g1_ref[...] + be1_ref[...]

    h2 = jnp.dot((ego * side), W2_ref[...], preferred_element_type=jnp.float32) + b2_ref[...]
    h2 = jnp.where(h2 > 0, h2, 0.01 * h2)
    mu2 = jnp.mean(h2, axis=-1, keepdims=True)
    var2 = jnp.mean((h2 - mu2) ** 2, axis=-1, keepdims=True)
    s2 = (h2 - mu2) * lax.rsqrt(var2 + 1e-5) * g2_ref[...] + be2_ref[...]

    out = s1 + s2
    if split_out:
        half = out.shape[-1] // 2
        ego_out_ref[0] = out[:, :half]
        ego_out_ref[1] = out[:, half:]
    else:
        ego_out_ref[...] = out
    nrm = jnp.sqrt(jnp.sum(out * out, axis=-1, keepdims=True))
    nrm_out_ref[...] = out / jnp.maximum(nrm, 1e-12)


def _dense_stage(ego, side2, W1, b1, W2, b2, g1, be1, g2, be2, split_out):
    din = ego.shape[1]
    dhalf = side2.shape[2]
    dout = W1.shape[1]
    grid = (N // ROW_BLK,)
    row_spec = pl.BlockSpec((ROW_BLK, din), lambda i: (i, 0))
    side_spec = pl.BlockSpec((2, ROW_BLK, dhalf), lambda i: (0, i, 0))
    out_spec = pl.BlockSpec((ROW_BLK, dout), lambda i: (i, 0))
    if split_out:
        ego_out_sds = jax.ShapeDtypeStruct((2, N, dout // 2), jnp.float32)
        ego_out_spec = pl.BlockSpec((2, ROW_BLK, dout // 2), lambda i: (0, i, 0))
    else:
        ego_out_sds = jax.ShapeDtypeStruct((N, dout), jnp.float32)
        ego_out_spec = out_spec
    full = lambda a: pl.BlockSpec(a.shape, lambda i: tuple(0 for _ in a.shape))
    vec = lambda a: pl.BlockSpec((1, dout), lambda i: (0, 0))
    ego_next, nrm = pl.pallas_call(
        functools.partial(_dense_body, split_out=split_out),
        grid=grid,
        in_specs=[row_spec, side_spec,
                  full(W1), vec(b1), full(W2), vec(b2),
                  vec(g1), vec(be1), vec(g2), vec(be2)],
        out_specs=[ego_out_spec, out_spec],
        out_shape=[ego_out_sds,
                   jax.ShapeDtypeStruct((N, dout), jnp.float32)],
    )(ego, side2, W1, b1.reshape(1, -1), W2, b2.reshape(1, -1),
      g1.reshape(1, -1), be1.reshape(1, -1), g2.reshape(1, -1), be2.reshape(1, -1))
    return ego_next, nrm


def kernel(x, edge_index, edge_weight,
           W1_0, b1_0, W2_0, b2_0, g1_0, be1_0, g2_0, be2_0,
           W1_1, b1_1, W2_1, b2_1, g1_1, be1_1, g2_1, be2_1):
    src2 = edge_index[1].reshape(E // CH, CH)
    dst2 = edge_index[0].reshape(E // CH, CH)
    w2 = edge_weight.reshape(E // CH, CH)

    zeros32 = jnp.zeros((N, 32), jnp.float32)
    zeros16 = jnp.zeros((N, 16), jnp.float32)

    x2 = jnp.transpose(x.reshape(N, 2, 32), (1, 0, 2))
    side0 = _agg_l0(x2[0], x2[1], src2, dst2, w2, zeros32)
    ego1, out1 = _dense_stage(x, side0, W1_0, b1_0, W2_0, b2_0,
                              g1_0, be1_0, g2_0, be2_0, split_out=True)

    side1 = _agg_l1(ego1[0], ego1[1], src2, dst2, w2, zeros16)
    ego2, out2 = _dense_stage(
        jnp.concatenate([ego1[0], ego1[1]], axis=1), side1,
        W1_1, b1_1, W2_1, b2_1, g1_1, be1_1, g2_1, be2_1, split_out=False)

    return jnp.concatenate([x, out1, out2], axis=1)


# speedup vs baseline: 5.6726x; 1.1850x over previous
import functools

import jax
import jax.numpy as jnp
from jax import lax
from jax.experimental import pallas as pl
from jax.experimental.pallas import tpu as pltpu
from jax.experimental.pallas import tpu_sc as plsc

N = 50000
E = 800000
ROW_BLK = 1000

NC = 2
NS = 16
CH = 80
CPS = 25
NSUP = 25
CHUNKS_PER_TILE = E // (NS * CH)

ROWS_MAIN = 3128
ROWS_LAST = N - 15 * ROWS_MAIN


def _scale_rows(rows_ref, w_ref, cc, dh):
    for g in range(CH // 16):
        wv = w_ref[cc, pl.ds(g * 16, 16)]
        for j in range(16):
            e = g * 16 + j
            w = wv[j]
            for h in range(dh // 16):
                r = rows_ref[e, pl.ds(h * 16, 16)]
                rows_ref[e, pl.ds(h * 16, 16)] = r * w


def _make_sc_agg(dh):
    mesh = plsc.VectorSubcoreMesh(core_axis_name="c", subcore_axis_name="s")

    @functools.partial(
        pl.kernel,
        out_type=jax.ShapeDtypeStruct((NC, N, dh), jnp.float32),
        mesh=mesh,
        scratch_types=[
            pltpu.VMEM((CPS, CH), jnp.int32),
            pltpu.VMEM((CPS, CH), jnp.int32),
            pltpu.VMEM((CPS, CH), jnp.float32),
            pltpu.VMEM((CH, dh), jnp.float32),
            pltpu.VMEM((CH, dh), jnp.float32),
            pltpu.VMEM_SHARED((N, dh), jnp.float32),
            pltpu.SemaphoreType.DMA,
            pltpu.SemaphoreType.DMA,
        ],
        compiler_params=pltpu.CompilerParams(use_tc_tiling_on_sc=False),
    )
    def agg(tabA, tabB, src2, dst2, w2, zeros, out,
            src_v, dst_v, w_v, rows0, rows1, acc, sem0, sem1):
        c = lax.axis_index("c")
        s = lax.axis_index("s")

        zbase = s * ROWS_MAIN

        @pl.when(s < 15)
        def _():
            pltpu.sync_copy(zeros.at[pl.ds(zbase, ROWS_MAIN)],
                            acc.at[pl.ds(zbase, ROWS_MAIN)])

        @pl.when(s == 15)
        def _():
            pltpu.sync_copy(zeros.at[pl.ds(15 * ROWS_MAIN, ROWS_LAST)],
                            acc.at[pl.ds(15 * ROWS_MAIN, ROWS_LAST)])

        plsc.subcore_barrier()

        tile_row = s * CHUNKS_PER_TILE
        rows = (rows0, rows1)
        sems = (sem0, sem1)

        def start_gather(cc, b):
            @pl.when(c == 0)
            def _():
                pltpu.make_async_copy(tabA.at[src_v.at[cc]], rows[b], sems[b]).start()

            @pl.when(c == 1)
            def _():
                pltpu.make_async_copy(tabB.at[src_v.at[cc]], rows[b], sems[b]).start()

        def wait_gather(b):
            pltpu.make_async_copy(tabA.at[src_v.at[0]], rows[b], sems[b]).wait()

        def finish(cc, b):
            _scale_rows(rows[b], w_v, cc, dh)
            pltpu.sync_copy(rows[b], acc.at[dst_v.at[cc]], add=True)

        def sup_body(si, carry):
            base = tile_row + si * CPS
            pltpu.sync_copy(src2.at[pl.ds(base, CPS)], src_v)
            pltpu.sync_copy(dst2.at[pl.ds(base, CPS)], dst_v)
            pltpu.sync_copy(w2.at[pl.ds(base, CPS)], w_v)

            start_gather(0, 0)

            def pair_body(k, carry2):
                cc = k * 2
                wait_gather(0)
                start_gather(cc + 1, 1)
                finish(cc, 0)
                wait_gather(1)
                start_gather(cc + 2, 0)
                finish(cc + 1, 1)
                return carry2

            lax.fori_loop(0, (CPS - 1) // 2, pair_body, 0)
            wait_gather(0)
            finish(CPS - 1, 0)
            return carry

        lax.fori_loop(0, NSUP, sup_body, 0)

        plsc.subcore_barrier()

        @pl.when(s < 15)
        def _():
            pltpu.sync_copy(acc.at[pl.ds(zbase, ROWS_MAIN)],
                            out.at[c, pl.ds(zbase, ROWS_MAIN)])

        @pl.when(s == 15)
        def _():
            pltpu.sync_copy(acc.at[pl.ds(15 * ROWS_MAIN, ROWS_LAST)],
                            out.at[c, pl.ds(15 * ROWS_MAIN, ROWS_LAST)])

    return agg


_agg_l0 = _make_sc_agg(32)
_agg_l1 = _make_sc_agg(16)


def _dense_body(ego_ref, side_ref, W1_ref, b1_ref, W2_ref, b2_ref,
                g1_ref, be1_ref, g2_ref, be2_ref,
                ego_out_ref, nrm_out_ref, *, split_out):
    ego = ego_ref[...]
    side = jnp.concatenate([side_ref[0], side_ref[1]], axis=-1)

    h1 = jnp.dot((ego + side), W1_ref[...], preferred_element_type=jnp.float32) + b1_ref[...]
    h1 = jnp.where(h1 > 0, h1, 0.01 * h1)
    mu1 = jnp.mean(h1, axis=-1, keepdims=True)
    var1 = jnp.mean((h1 - mu1) ** 2, axis=-1, keepdims=True)
    s1 = (h1 - mu1) * lax.rsqrt(var1 + 1e-5) * g1_ref[...] + be1_ref[...]

    h2 = jnp.dot((ego * side), W2_ref[...], preferred_element_type=jnp.float32) + b2_ref[...]
    h2 = jnp.where(h2 > 0, h2, 0.01 * h2)
    mu2 = jnp.mean(h2, axis=-1, keepdims=True)
    var2 = jnp.mean((h2 - mu2) ** 2, axis=-1, keepdims=True)
    s2 = (h2 - mu2) * lax.rsqrt(var2 + 1e-5) * g2_ref[...] + be2_ref[...]

    out = s1 + s2
    if split_out:
        half = out.shape[-1] // 2
        ego_out_ref[0] = out[:, :half]
        ego_out_ref[1] = out[:, half:]
    else:
        ego_out_ref[...] = out
    nrm = jnp.sqrt(jnp.sum(out * out, axis=-1, keepdims=True))
    nrm_out_ref[...] = out / jnp.maximum(nrm, 1e-12)


def _dense_stage(ego, side2, W1, b1, W2, b2, g1, be1, g2, be2, split_out):
    din = ego.shape[1]
    dhalf = side2.shape[2]
    dout = W1.shape[1]
    grid = (N // ROW_BLK,)
    row_spec = pl.BlockSpec((ROW_BLK, din), lambda i: (i, 0))
    side_spec = pl.BlockSpec((2, ROW_BLK, dhalf), lambda i: (0, i, 0))
    out_spec = pl.BlockSpec((ROW_BLK, dout), lambda i: (i, 0))
    if split_out:
        ego_out_sds = jax.ShapeDtypeStruct((2, N, dout // 2), jnp.float32)
        ego_out_spec = pl.BlockSpec((2, ROW_BLK, dout // 2), lambda i: (0, i, 0))
    else:
        ego_out_sds = jax.ShapeDtypeStruct((N, dout), jnp.float32)
        ego_out_spec = out_spec
    full = lambda a: pl.BlockSpec(a.shape, lambda i: tuple(0 for _ in a.shape))
    vec = lambda a: pl.BlockSpec((1, dout), lambda i: (0, 0))
    ego_next, nrm = pl.pallas_call(
        functools.partial(_dense_body, split_out=split_out),
        grid=grid,
        in_specs=[row_spec, side_spec,
                  full(W1), vec(b1), full(W2), vec(b2),
                  vec(g1), vec(be1), vec(g2), vec(be2)],
        out_specs=[ego_out_spec, out_spec],
        out_shape=[ego_out_sds,
                   jax.ShapeDtypeStruct((N, dout), jnp.float32)],
    )(ego, side2, W1, b1.reshape(1, -1), W2, b2.reshape(1, -1),
      g1.reshape(1, -1), be1.reshape(1, -1), g2.reshape(1, -1), be2.reshape(1, -1))
    return ego_next, nrm


def kernel(x, edge_index, edge_weight,
           W1_0, b1_0, W2_0, b2_0, g1_0, be1_0, g2_0, be2_0,
           W1_1, b1_1, W2_1, b2_1, g1_1, be1_1, g2_1, be2_1):
    src2 = edge_index[1].reshape(E // CH, CH)
    dst2 = edge_index[0].reshape(E // CH, CH)
    w2 = edge_weight.reshape(E // CH, CH)

    zeros32 = jnp.zeros((N, 32), jnp.float32)
    zeros16 = jnp.zeros((N, 16), jnp.float32)

    x2 = jnp.transpose(x.reshape(N, 2, 32), (1, 0, 2))
    side0 = _agg_l0(x2[0], x2[1], src2, dst2, w2, zeros32)
    ego1, out1 = _dense_stage(x, side0, W1_0, b1_0, W2_0, b2_0,
                              g1_0, be1_0, g2_0, be2_0, split_out=True)

    side1 = _agg_l1(ego1[0], ego1[1], src2, dst2, w2, zeros16)
    ego2, out2 = _dense_stage(
        jnp.concatenate([ego1[0], ego1[1]], axis=1), side1,
        W1_1, b1_1, W2_1, b2_1, g1_1, be1_1, g2_1, be2_1, split_out=False)

    return jnp.concatenate([x, out1, out2], axis=1)
